# baseline (speedup 1.0000x reference)
import jax
import jax.numpy as jnp
from jax.experimental import pallas as pl
from jax.experimental.pallas import tpu as pltpu

_FOLD_CHUNKS = 8


def _kernel_body(w2_ref, b1_ref, b2_ref, w1_hbm, x_ref, out_ref,
                 wc_ref, bc_ref, w1_buf, dma_sems):
    D = wc_ref.shape[0]
    rows = D // _FOLD_CHUNKS

    @pl.when(pl.program_id(0) == 0)
    def _fold_weights():
        def cp(k, slot):
            return pltpu.make_async_copy(
                w1_hbm.at[pl.ds(k * rows, rows), :],
                w1_buf.at[slot],
                dma_sems.at[slot])

        cp(0, 0).start()
        for k in range(_FOLD_CHUNKS):
            if k + 1 < _FOLD_CHUNKS:
                cp(k + 1, (k + 1) % 2).start()
            cp(k, k % 2).wait()
            wc_ref[k * rows:(k + 1) * rows, :] = jnp.dot(
                w1_buf[k % 2], w2_ref[...],
                preferred_element_type=jnp.float32).astype(jnp.bfloat16)
        bc = jnp.dot(b1_ref[...], w2_ref[...],
                     preferred_element_type=jnp.float32) + b2_ref[...]
        bc_ref[...] = jnp.broadcast_to(bc, bc_ref.shape)

    y = jnp.dot(x_ref[...].astype(jnp.bfloat16), wc_ref[...],
                preferred_element_type=jnp.float32)
    y = y + bc_ref[0:1, :]
    m = jnp.max(y, axis=1, keepdims=True)
    e = jnp.exp(y - m)
    out_ref[...] = (e / jnp.sum(e, axis=1, keepdims=True)).astype(out_ref.dtype)


def _forward(x, w1, b1_2d, w2, b2_2d, tile_b):
    B, D = x.shape
    H = w1.shape[1]
    O = w2.shape[1]
    n_tiles = B // tile_b
    return pl.pallas_call(
        _kernel_body,
        out_shape=jax.ShapeDtypeStruct((B, O), jnp.float32),
        grid=(n_tiles,),
        in_specs=[
            pl.BlockSpec((H, O), lambda i: (0, 0)),
            pl.BlockSpec((1, H), lambda i: (0, 0)),
            pl.BlockSpec((1, O), lambda i: (0, 0)),
            pl.BlockSpec(memory_space=pltpu.MemorySpace.HBM),
            pl.BlockSpec((tile_b, D), lambda i: (i, 0)),
        ],
        out_specs=pl.BlockSpec((tile_b, O), lambda i: (i, 0)),
        scratch_shapes=[
            pltpu.VMEM((D, O), jnp.bfloat16),
            pltpu.VMEM((8, O), jnp.float32),
            pltpu.VMEM((2, D // _FOLD_CHUNKS, H), jnp.float32),
            pltpu.SemaphoreType.DMA((2,)),
        ],
        compiler_params=pltpu.CompilerParams(
            dimension_semantics=("arbitrary",),
            vmem_limit_bytes=64 << 20,
        ),
    )(w2, b1_2d, b2_2d, w1, x)


def kernel(x, w1, b1, w2, b2):
    B, D = x.shape
    H = w1.shape[1]
    O = w2.shape[1]
    b1_2d = b1.reshape(1, H)
    b2_2d = b2.reshape(1, O)

    tile_b = 1024
    while B % tile_b != 0 or (B // tile_b) % 2 != 0:
        tile_b //= 2
    return _forward(x, w1, b1_2d, w2, b2_2d, tile_b)


# speedup vs baseline: 1.0221x; 1.0221x over previous
import jax
import jax.numpy as jnp
from jax.experimental import pallas as pl
from jax.experimental.pallas import tpu as pltpu

_FOLD_CHUNKS = 4


def _kernel_body(w2_ref, b1_ref, b2_ref, w1_hbm, x_ref, out_ref,
                 wc_ref, bc_ref, w1_buf, dma_sems):
    D = wc_ref.shape[0]
    rows = D // _FOLD_CHUNKS

    @pl.when(pl.program_id(0) == 0)
    def _fold_weights():
        def cp(k, slot):
            return pltpu.make_async_copy(
                w1_hbm.at[pl.ds(k * rows, rows), :],
                w1_buf.at[slot],
                dma_sems.at[slot])

        cp(0, 0).start()
        for k in range(_FOLD_CHUNKS):
            if k + 1 < _FOLD_CHUNKS:
                cp(k + 1, (k + 1) % 2).start()
            cp(k, k % 2).wait()
            wc_ref[k * rows:(k + 1) * rows, :] = jnp.dot(
                w1_buf[k % 2], w2_ref[...],
                preferred_element_type=jnp.float32).astype(jnp.bfloat16)
        bc = jnp.dot(b1_ref[...], w2_ref[...],
                     preferred_element_type=jnp.float32) + b2_ref[...]
        bc_ref[...] = jnp.broadcast_to(bc, bc_ref.shape)

    tb = x_ref.shape[0]
    half = tb // 2
    xh = x_ref[...].astype(jnp.bfloat16)
    ys = [jnp.dot(xh[h * half:(h + 1) * half], wc_ref[...],
                  preferred_element_type=jnp.float32) + bc_ref[0:1, :]
          for h in range(2)]
    for h, y in enumerate(ys):
        m = jnp.max(y, axis=1, keepdims=True)
        e = jnp.exp(y - m)
        out_ref[h * half:(h + 1) * half, :] = (
            e / jnp.sum(e, axis=1, keepdims=True)).astype(out_ref.dtype)


def _forward(x, w1, b1_2d, w2, b2_2d, tile_b):
    B, D = x.shape
    H = w1.shape[1]
    O = w2.shape[1]
    n_tiles = B // tile_b
    return pl.pallas_call(
        _kernel_body,
        out_shape=jax.ShapeDtypeStruct((B, O), jnp.float32),
        grid=(n_tiles,),
        in_specs=[
            pl.BlockSpec((H, O), lambda i: (0, 0)),
            pl.BlockSpec((1, H), lambda i: (0, 0)),
            pl.BlockSpec((1, O), lambda i: (0, 0)),
            pl.BlockSpec(memory_space=pltpu.MemorySpace.HBM),
            pl.BlockSpec((tile_b, D), lambda i: (i, 0)),
        ],
        out_specs=pl.BlockSpec((tile_b, O), lambda i: (i, 0)),
        scratch_shapes=[
            pltpu.VMEM((D, O), jnp.bfloat16),
            pltpu.VMEM((8, O), jnp.float32),
            pltpu.VMEM((2, D // _FOLD_CHUNKS, H), jnp.float32),
            pltpu.SemaphoreType.DMA((2,)),
        ],
        compiler_params=pltpu.CompilerParams(
            dimension_semantics=("arbitrary",),
            vmem_limit_bytes=64 << 20,
        ),
    )(w2, b1_2d, b2_2d, w1, x)


def kernel(x, w1, b1, w2, b2):
    B, D = x.shape
    H = w1.shape[1]
    O = w2.shape[1]
    b1_2d = b1.reshape(1, H)
    b2_2d = b2.reshape(1, O)

    tile_b = 1024
    while B % tile_b != 0 or (B // tile_b) % 2 != 0:
        tile_b //= 2
    return _forward(x, w1, b1_2d, w2, b2_2d, tile_b)


# speedup vs baseline: 1.0292x; 1.0069x over previous
import jax
import jax.numpy as jnp
from jax.experimental import pallas as pl
from jax.experimental.pallas import tpu as pltpu

_FOLD_CHUNKS = 4


def _kernel_body(w2_ref, b1_ref, b2_ref, w1_hbm, x_ref, out_ref,
                 wc_ref, bc_ref, w1_buf, dma_sems):
    D = wc_ref.shape[0]
    rows = D // _FOLD_CHUNKS

    @pl.when(pl.program_id(0) == 0)
    def _fold_weights():
        def cp(k, slot):
            return pltpu.make_async_copy(
                w1_hbm.at[pl.ds(k * rows, rows), :],
                w1_buf.at[slot],
                dma_sems.at[slot])

        cp(0, 0).start()
        bc = jnp.dot(b1_ref[...], w2_ref[...],
                     preferred_element_type=jnp.float32) + b2_ref[...]
        bc_ref[...] = jnp.broadcast_to(bc, bc_ref.shape)
        for k in range(_FOLD_CHUNKS):
            if k + 1 < _FOLD_CHUNKS:
                cp(k + 1, (k + 1) % 2).start()
            cp(k, k % 2).wait()
            wc_ref[k * rows:(k + 1) * rows, :] = jnp.dot(
                w1_buf[k % 2], w2_ref[...],
                preferred_element_type=jnp.float32).astype(jnp.bfloat16)

    tb = x_ref.shape[0]
    half = tb // 2
    xh = x_ref[...].astype(jnp.bfloat16)
    ys = [jnp.dot(xh[h * half:(h + 1) * half], wc_ref[...],
                  preferred_element_type=jnp.float32) + bc_ref[0:1, :]
          for h in range(2)]
    for h, y in enumerate(ys):
        m = jnp.max(y, axis=1, keepdims=True)
        e = jnp.exp(y - m)
        out_ref[h * half:(h + 1) * half, :] = (
            e / jnp.sum(e, axis=1, keepdims=True)).astype(out_ref.dtype)


def _forward(x, w1, b1_2d, w2, b2_2d, tile_b):
    B, D = x.shape
    H = w1.shape[1]
    O = w2.shape[1]
    n_tiles = B // tile_b
    return pl.pallas_call(
        _kernel_body,
        out_shape=jax.ShapeDtypeStruct((B, O), jnp.float32),
        grid=(n_tiles,),
        in_specs=[
            pl.BlockSpec((H, O), lambda i: (0, 0)),
            pl.BlockSpec((1, H), lambda i: (0, 0)),
            pl.BlockSpec((1, O), lambda i: (0, 0)),
            pl.BlockSpec(memory_space=pltpu.MemorySpace.HBM),
            pl.BlockSpec((tile_b, D), lambda i: (i, 0)),
        ],
        out_specs=pl.BlockSpec((tile_b, O), lambda i: (i, 0)),
        scratch_shapes=[
            pltpu.VMEM((D, O), jnp.bfloat16),
            pltpu.VMEM((8, O), jnp.float32),
            pltpu.VMEM((2, D // _FOLD_CHUNKS, H), jnp.float32),
            pltpu.SemaphoreType.DMA((2,)),
        ],
        compiler_params=pltpu.CompilerParams(
            dimension_semantics=("arbitrary",),
            vmem_limit_bytes=64 << 20,
        ),
    )(w2, b1_2d, b2_2d, w1, x)


def kernel(x, w1, b1, w2, b2):
    B, D = x.shape
    H = w1.shape[1]
    O = w2.shape[1]
    b1_2d = b1.reshape(1, H)
    b2_2d = b2.reshape(1, O)

    tile_b = 1024
    while B % tile_b != 0 or (B // tile_b) % 2 != 0:
        tile_b //= 2
    return _forward(x, w1, b1_2d, w2, b2_2d, tile_b)


# speedup vs baseline: 1.0508x; 1.0210x over previous
import jax
import jax.numpy as jnp
from jax.experimental import pallas as pl
from jax.experimental.pallas import tpu as pltpu

_FOLD_CHUNKS = 4


def _kernel_body(w2_ref, b1_ref, b2_ref, w1_hbm, x_ref, out_ref,
                 wc_ref, bc_ref, w1_buf, dma_sems):
    D = wc_ref.shape[0]
    rows = D // _FOLD_CHUNKS

    @pl.when(pl.program_id(0) == 0)
    def _fold_weights():
        def cp(k, slot):
            return pltpu.make_async_copy(
                w1_hbm.at[pl.ds(k * rows, rows), :],
                w1_buf.at[slot],
                dma_sems.at[slot])

        cp(0, 0).start()
        bc = jnp.dot(b1_ref[...], w2_ref[...],
                     preferred_element_type=jnp.float32) + b2_ref[...]
        bc_ref[...] = jnp.broadcast_to(bc, bc_ref.shape)
        for k in range(_FOLD_CHUNKS):
            if k + 1 < _FOLD_CHUNKS:
                cp(k + 1, (k + 1) % 2).start()
            cp(k, k % 2).wait()
            wc_ref[k * rows:(k + 1) * rows, :] = jnp.dot(
                w1_buf[k % 2], w2_ref[...],
                preferred_element_type=jnp.float32).astype(jnp.bfloat16)

    tb = x_ref.shape[0]
    nsub = 4
    sub = tb // nsub
    xh = x_ref[...].astype(jnp.bfloat16)
    ys = [jnp.dot(xh[h * sub:(h + 1) * sub], wc_ref[...],
                  preferred_element_type=jnp.float32) + bc_ref[0:1, :]
          for h in range(nsub)]
    for h, y in enumerate(ys):
        m = jnp.max(y, axis=1, keepdims=True)
        e = jnp.exp(y - m)
        out_ref[h * sub:(h + 1) * sub, :] = (
            e / jnp.sum(e, axis=1, keepdims=True)).astype(out_ref.dtype)


def _forward(x, w1, b1_2d, w2, b2_2d, tile_b):
    B, D = x.shape
    H = w1.shape[1]
    O = w2.shape[1]
    n_tiles = B // tile_b
    return pl.pallas_call(
        _kernel_body,
        out_shape=jax.ShapeDtypeStruct((B, O), jnp.float32),
        grid=(n_tiles,),
        in_specs=[
            pl.BlockSpec((H, O), lambda i: (0, 0)),
            pl.BlockSpec((1, H), lambda i: (0, 0)),
            pl.BlockSpec((1, O), lambda i: (0, 0)),
            pl.BlockSpec(memory_space=pltpu.MemorySpace.HBM),
            pl.BlockSpec((tile_b, D), lambda i: (i, 0)),
        ],
        out_specs=pl.BlockSpec((tile_b, O), lambda i: (i, 0)),
        scratch_shapes=[
            pltpu.VMEM((D, O), jnp.bfloat16),
            pltpu.VMEM((8, O), jnp.float32),
            pltpu.VMEM((2, D // _FOLD_CHUNKS, H), jnp.float32),
            pltpu.SemaphoreType.DMA((2,)),
        ],
        compiler_params=pltpu.CompilerParams(
            dimension_semantics=("arbitrary",),
            vmem_limit_bytes=64 << 20,
        ),
    )(w2, b1_2d, b2_2d, w1, x)


def kernel(x, w1, b1, w2, b2):
    B, D = x.shape
    H = w1.shape[1]
    O = w2.shape[1]
    b1_2d = b1.reshape(1, H)
    b2_2d = b2.reshape(1, O)

    tile_b = 1024
    while B % tile_b != 0 or (B // tile_b) % 2 != 0:
        tile_b //= 2
    return _forward(x, w1, b1_2d, w2, b2_2d, tile_b)
